# baseline (speedup 1.0000x reference)
import jax
import jax.numpy as jnp
from jax.experimental import pallas as pl
from jax.experimental.pallas import tpu as pltpu
from jax.experimental.pallas import tpu_sc as plsc

F = 128
NCORES = 2
NSUB = 16
CPW = 64
BPW = 8
LANES = 16


def kernel(mask, row_embed, col_embed):
    b, h, w = mask.shape
    mesh = plsc.VectorSubcoreMesh(core_axis_name="c", subcore_axis_name="s")

    @pl.kernel(
        out_type=jax.ShapeDtypeStruct((b, 2 * F, h, w), jnp.float32),
        mesh=mesh,
        compiler_params=pltpu.CompilerParams(
            use_tc_tiling_on_sc=False, needs_layout_passes=False
        ),
        scratch_types=[
            pltpu.VMEM((CPW, h, w), jnp.float32),
            pltpu.VMEM((h, F), jnp.float32),
            pltpu.SemaphoreType.DMA,
        ],
    )
    def sc_kernel(row_hbm, col_hbm, out_hbm, chunk, tab, sem):
        core = jax.lax.axis_index("c")
        sub = jax.lax.axis_index("s")
        wid = core * NSUB + sub
        q = wid // 8
        g = wid % 8
        c0 = q * CPW
        iota = jax.lax.iota(jnp.int32, LANES)

        @pl.when(q < 2)
        def _x_half():
            pltpu.async_copy(col_hbm.at[pl.ds(0, w)], tab, sem).wait()

            @pl.loop(0, CPW)
            def _xj(j):
                cvec = jnp.zeros((LANES,), jnp.int32) + (c0 + j)
                v0 = plsc.load_gather(tab, [iota, cvec])
                v1 = plsc.load_gather(tab, [iota + LANES, cvec])

                @pl.loop(0, h)
                def _xy(y):
                    chunk.at[j].at[y][pl.ds(0, LANES)] = v0
                    chunk.at[j].at[y][pl.ds(LANES, LANES)] = v1

        @pl.when(q >= 2)
        def _y_half():
            pltpu.async_copy(row_hbm.at[pl.ds(0, h)], tab, sem).wait()

            @pl.loop(0, CPW)
            def _yj(j):
                ccvec = jnp.zeros((LANES,), jnp.int32) + (c0 + j - F)

                @pl.loop(0, h)
                def _yy(y):
                    yvec = jnp.zeros((LANES,), jnp.int32) + y
                    v = plsc.load_gather(tab, [yvec, ccvec])
                    chunk.at[j].at[y][pl.ds(0, LANES)] = v
                    chunk.at[j].at[y][pl.ds(LANES, LANES)] = v

        copies = [
            pltpu.make_async_copy(
                chunk, out_hbm.at[g * BPW + k, pl.ds(c0, CPW)], sem
            )
            for k in range(BPW)
        ]
        for c in copies:
            c.start()
        for c in copies:
            c.wait()

    return sc_kernel(row_embed, col_embed)


# speedup vs baseline: 3.6188x; 3.6188x over previous
import jax
import jax.numpy as jnp
from jax.experimental import pallas as pl
from jax.experimental.pallas import tpu as pltpu

NUM_POS_FEATS = 128
REP = 8
NSEM = 8


def _pos_body(row_ref, col_ref, out_ref, stage_ref, sems):
    f = NUM_POS_FEATS
    h = row_ref.shape[0]
    w = col_ref.shape[0]
    b = out_ref.shape[0]
    colT = col_ref[...].T
    rowT = row_ref[...].T
    xp = jnp.tile(colT, (1, h))
    yp = jnp.repeat(rowT, w, axis=1)
    plane = jnp.concatenate([xp, yp], axis=0)
    stage_ref[...] = jnp.broadcast_to(plane[None], (REP, 2 * f, h * w))
    copies = [
        pltpu.make_async_copy(
            stage_ref.at[i % REP],
            out_ref.at[i],
            sems.at[i % NSEM],
        )
        for i in range(b)
    ]
    for c in copies:
        c.start()
    for c in copies:
        c.wait()


def kernel(mask, row_embed, col_embed):
    b, h, w = mask.shape
    f = NUM_POS_FEATS
    out = pl.pallas_call(
        _pos_body,
        in_specs=[
            pl.BlockSpec((h, f), lambda: (0, 0)),
            pl.BlockSpec((w, f), lambda: (0, 0)),
        ],
        out_specs=pl.BlockSpec(memory_space=pltpu.MemorySpace.HBM),
        out_shape=jax.ShapeDtypeStruct((b, 2 * f, h * w), jnp.float32),
        scratch_shapes=[
            pltpu.VMEM((REP, 2 * f, h * w), jnp.float32),
            pltpu.SemaphoreType.DMA((NSEM,)),
        ],
    )(row_embed[:h], col_embed[:w])
    return out.reshape(b, 2 * f, h, w)
